# baseline (speedup 1.0000x reference)
import functools

import jax
import jax.numpy as jnp
from jax import lax
from jax.experimental import pallas as pl
from jax.experimental.pallas import tpu as pltpu
from jax.experimental.pallas import tpu_sc as plsc

N_NODES = 10000
N_PAD = 10240
E_EDGES = 100000
E_PAD = 102400
NGRAPH = 64
DUMMY = 10000
NBLK = 512
N_BLOCKS = N_PAD // NBLK
E_BLOCKS = E_PAD // NBLK
DR = 384

NC, NS = 2, 16
NW = NC * NS
KCH = 128
EW = E_PAD // NW
GCH = EW // KCH
ET = E_PAD // NS
SCH = ET // KCH
NRT = N_PAD // NS

_INTERPRET = False


def _elu(v):
    return jnp.where(v > 0, v, jnp.exp(jnp.minimum(v, 0.0)) - 1.0)


def _dot(a, b):
    return jnp.dot(a, b, preferred_element_type=jnp.float32)


def _stats_x_body(x_ref, o_ref):
    x = x_ref[...]
    m = jnp.mean(x, axis=0, keepdims=True)
    v = jnp.mean((x - m) ** 2, axis=0, keepdims=True)
    o_ref[...] = jnp.concatenate([m, v], axis=0)


def _stats_x(x):
    d = x.shape[1]
    return pl.pallas_call(
        _stats_x_body,
        out_shape=jax.ShapeDtypeStruct((2, d), jnp.float32),
        interpret=_INTERPRET,
    )(x)


def _stats_e_body(e_ref, o_ref, s_ref, q_ref):
    i = pl.program_id(0)
    e = e_ref[...]
    ps = jnp.sum(e, axis=0, keepdims=True)
    pq = jnp.sum(e * e, axis=0, keepdims=True)

    @pl.when(i == 0)
    def _():
        s_ref[...] = ps
        q_ref[...] = pq

    @pl.when(i > 0)
    def _():
        s_ref[...] += ps
        q_ref[...] += pq

    @pl.when(i == pl.num_programs(0) - 1)
    def _():
        m = s_ref[...] / E_EDGES
        v = q_ref[...] / E_EDGES - m * m
        o_ref[...] = jnp.concatenate([m, v], axis=0)


def _stats_e(e):
    d = e.shape[1]
    nb = e.shape[0] // 4000
    return pl.pallas_call(
        _stats_e_body,
        grid=(nb,),
        in_specs=[pl.BlockSpec((4000, d), lambda i: (i, 0))],
        out_specs=pl.BlockSpec((2, d), lambda i: (0, 0)),
        out_shape=jax.ShapeDtypeStruct((2, d), jnp.float32),
        scratch_shapes=[pltpu.VMEM((1, d), jnp.float32),
                        pltpu.VMEM((1, d), jnp.float32)],
        interpret=_INTERPRET,
    )(e)


def _bn_body(x_ref, st_ref, g_ref, b_ref, o_ref):
    st = st_ref[...]
    m, v = st[0:1], st[1:2]
    o_ref[...] = (x_ref[...] - m) / jnp.sqrt(v + 1e-5) * g_ref[...] + b_ref[...]


def _bn_apply(x, stats, g, b, blk):
    n, d = x.shape
    return pl.pallas_call(
        _bn_body,
        grid=(n // blk,),
        in_specs=[pl.BlockSpec((blk, d), lambda i: (i, 0)),
                  pl.BlockSpec((2, d), lambda i: (0, 0)),
                  pl.BlockSpec((1, d), lambda i: (0, 0)),
                  pl.BlockSpec((1, d), lambda i: (0, 0))],
        out_specs=pl.BlockSpec((blk, d), lambda i: (i, 0)),
        out_shape=jax.ShapeDtypeStruct((n, d), jnp.float32),
        interpret=_INTERPRET,
    )(x, stats, g.reshape(1, d), b.reshape(1, d))


def _p2_body(e_ref, g1_ref, g2_ref, wrn_ref, wc_ref,
             wee_ref, b1e_ref, w2e_ref, b2e_ref,
             wh_ref, b1n_ref, w2n_ref, b2n_ref, en_ref, h_ref):
    t1 = _dot(g1_ref[...], wrn_ref[...])
    h1 = _elu(t1[:, :128] + _dot(g2_ref[...], wc_ref[...])
              + _dot(e_ref[...], wee_ref[...]) + b1e_ref[...])
    e_new = _dot(h1, w2e_ref[...]) + b2e_ref[...]
    en_ref[...] = e_new
    n1 = _elu(t1[:, 128:] + _dot(e_new, wh_ref[...]) + b1n_ref[...])
    h = _dot(n1, w2n_ref[...]) + b2n_ref[...]
    h_ref[0, :, :] = h[:, :128]
    h_ref[1, :, :] = h[:, 128:]


def _p2(e, g1, g2, wrn, wc, wee, b1e, w2e, b2e, wh, b1n, w2n, b2n):
    ne = e.shape[1]
    xw = g1.shape[1]
    eo = w2e.shape[1]
    full = lambda a: pl.BlockSpec(a.shape, lambda i: tuple(0 for _ in a.shape))
    return pl.pallas_call(
        _p2_body,
        grid=(E_BLOCKS,),
        in_specs=[pl.BlockSpec((NBLK, ne), lambda i: (i, 0)),
                  pl.BlockSpec((NBLK, xw), lambda i: (i, 0)),
                  pl.BlockSpec((NBLK, xw), lambda i: (i, 0)),
                  full(wrn), full(wc),
                  full(wee), full(b1e), full(w2e), full(b2e),
                  full(wh), full(b1n), full(w2n), full(b2n)],
        out_specs=[pl.BlockSpec((NBLK, eo), lambda i: (i, 0)),
                   pl.BlockSpec((2, NBLK, 128), lambda i: (0, i, 0))],
        out_shape=[jax.ShapeDtypeStruct((E_PAD, eo), jnp.float32),
                   jax.ShapeDtypeStruct((2, E_PAD, 128), jnp.float32)],
        interpret=_INTERPRET,
    )(e, g1, g2, wrn, wc, wee, b1e, w2e, b2e, wh, b1n, w2n, b2n)


def _p3_body(has_u, no, go, *refs):
    if has_u:
        (xn_ref, s0_ref, s1_ref, cnt_ref, b_ref, u_ref,
         wx2_ref, wh2a_ref, wh2b_ref, b1_ref, w2_ref, b2_ref,
         wg1u_ref, wg1g_ref, bg1_ref, wg2_ref, bg2_ref,
         xo_ref, uo_ref, gm_ref, gc_ref) = refs
    else:
        (xn_ref, s0_ref, s1_ref, cnt_ref, b_ref,
         wx2_ref, wh2a_ref, wh2b_ref, b1_ref, w2_ref, b2_ref,
         wg1g_ref, bg1_ref, wg2_ref, bg2_ref,
         xo_ref, uo_ref, gm_ref, gc_ref) = refs
    i = pl.program_id(0)
    cnt = jnp.maximum(cnt_ref[...], 1.0)
    hm0 = s0_ref[0, :, :] / cnt
    hm1 = s1_ref[0, :, :] / cnt
    t = _elu(_dot(xn_ref[...], wx2_ref[...]) + _dot(hm0, wh2a_ref[...])
             + _dot(hm1, wh2b_ref[...]) + b1_ref[...])
    xnew = _dot(t, w2_ref[...]) + b2_ref[...]
    xo_ref[...] = xnew

    gids = lax.broadcasted_iota(jnp.int32, (1, NGRAPH), 1)
    oh = (b_ref[...] == gids).astype(jnp.float32)
    part = lax.dot_general(oh, xnew, (((0,), (0,)), ((), ())),
                           preferred_element_type=jnp.float32)
    ones = jnp.ones((NBLK, 1), jnp.float32)
    pcnt = lax.dot_general(oh, ones, (((0,), (0,)), ((), ())),
                           preferred_element_type=jnp.float32)

    @pl.when(i == 0)
    def _():
        gm_ref[...] = part
        gc_ref[...] = pcnt

    @pl.when(i > 0)
    def _():
        gm_ref[...] += part
        gc_ref[...] += pcnt

    @pl.when(i == pl.num_programs(0) - 1)
    def _():
        gmean = gm_ref[...] / jnp.maximum(gc_ref[...], 1.0)
        pre = _dot(gmean, wg1g_ref[...]) + bg1_ref[...]
        if has_u:
            pre = pre + _dot(u_ref[...], wg1u_ref[...])
        ug1 = _elu(pre)
        uo_ref[...] = _dot(ug1, wg2_ref[...]) + bg2_ref[...]


def _p3(xn, S, cnt2d, batch2d, u, wx2, wh2a, wh2b, b1, w2, b2,
        wg1u, wg1g, bg1, wg2, bg2):
    has_u = u is not None
    nf = xn.shape[1]
    no = w2.shape[1]
    go = wg2.shape[1]
    full = lambda a: pl.BlockSpec(a.shape, lambda i: tuple(0 for _ in a.shape))
    in_specs = [pl.BlockSpec((NBLK, nf), lambda i: (i, 0)),
                pl.BlockSpec((1, NBLK, 128), lambda i: (0, i, 0)),
                pl.BlockSpec((1, NBLK, 128), lambda i: (1, i, 0)),
                pl.BlockSpec((NBLK, 1), lambda i: (i, 0)),
                pl.BlockSpec((NBLK, 1), lambda i: (i, 0))]
    args = [xn, S, S, cnt2d, batch2d]
    if has_u:
        in_specs.append(full(u))
        args.append(u)
    wargs = [wx2, wh2a, wh2b, b1, w2, b2]
    if has_u:
        wargs.append(wg1u)
    wargs += [wg1g, bg1, wg2, bg2]
    in_specs += [full(a) for a in wargs]
    args += wargs
    body = functools.partial(_p3_body, has_u, no, go)
    return pl.pallas_call(
        body,
        grid=(N_BLOCKS,),
        in_specs=in_specs,
        out_specs=[pl.BlockSpec((NBLK, no), lambda i: (i, 0)),
                   pl.BlockSpec((NGRAPH, go), lambda i: (0, 0))],
        out_shape=[jax.ShapeDtypeStruct((N_PAD, no), jnp.float32),
                   jax.ShapeDtypeStruct((NGRAPH, go), jnp.float32)],
        scratch_shapes=[pltpu.VMEM((NGRAPH, no), jnp.float32),
                        pltpu.VMEM((NGRAPH, 1), jnp.float32)],
        interpret=_INTERPRET,
    )(*args)


def _p5_body(u_ref, w1_ref, b1_ref, w2_ref, b2_ref, o_ref):
    y1 = _elu(_dot(u_ref[...], w1_ref[...]) + b1_ref[...])
    o_ref[...] = _dot(y1, w2_ref[...]) + b2_ref[...]


def _p5(u, w1, b1, w2p, b2p, cblk):
    cpad = w2p.shape[1]
    full = lambda a: pl.BlockSpec(a.shape, lambda j: tuple(0 for _ in a.shape))
    return pl.pallas_call(
        _p5_body,
        grid=(cpad // cblk,),
        in_specs=[full(u), full(w1), full(b1),
                  pl.BlockSpec((w2p.shape[0], cblk), lambda j: (0, j)),
                  pl.BlockSpec((1, cblk), lambda j: (0, j))],
        out_specs=pl.BlockSpec((NGRAPH, cblk), lambda j: (0, j)),
        out_shape=jax.ShapeDtypeStruct((NGRAPH, cpad), jnp.float32),
        interpret=_INTERPRET,
    )(u, w1, b1, w2p, b2p)


def _sc_gather_body(xtab, ridx, cidx, g1, g2, ri_v, ci_v, b1_v, b2_v, s1, s2):
    wid = lax.axis_index("s") * NC + lax.axis_index("c")
    pltpu.sync_copy(ridx.at[wid], ri_v)
    pltpu.sync_copy(cidx.at[wid], ci_v)

    def chunk(j, carry):
        eb = wid * EW + j * KCH
        cp1 = pltpu.async_copy(xtab.at[ri_v.at[j]], b1_v, s1)
        cp2 = pltpu.async_copy(xtab.at[ci_v.at[j]], b2_v, s2)
        cp1.wait()
        cp2.wait()
        pltpu.sync_copy(b1_v, g1.at[pl.ds(eb, KCH)])
        pltpu.sync_copy(b2_v, g2.at[pl.ds(eb, KCH)])
        return carry

    lax.fori_loop(0, GCH, chunk, 0)


def _sc_scatter_body(with_cnt, *refs):
    if with_cnt:
        (h_st, cidx, zer2, zer1, s_out, cnt_out,
         idx_v, h_v, ones_v, acc, cnt_sh) = refs
    else:
        (h_st, cidx, zer2, s_out, idx_v, h_v, acc) = refs
    c = lax.axis_index("c")
    s = lax.axis_index("s")
    rb = s * NRT
    pltpu.sync_copy(zer2.at[pl.ds(rb, NRT)], acc.at[pl.ds(rb, NRT)])
    if with_cnt:
        @pl.when(c == 0)
        def _():
            pltpu.sync_copy(zer1.at[pl.ds(rb, NRT)], cnt_sh.at[pl.ds(rb, NRT)])
        for l in range(KCH // 16):
            ones_v[pl.ds(l * 16, 16)] = jnp.ones((16,), jnp.float32)
    plsc.subcore_barrier()

    pltpu.sync_copy(cidx.at[s], idx_v)

    def chunk(j, carry):
        eb = s * ET + j * KCH
        pltpu.sync_copy(h_st.at[c, pl.ds(eb, KCH)], h_v)
        pltpu.sync_copy(h_v, acc.at[idx_v.at[j]], add=True)
        if with_cnt:
            @pl.when(c == 0)
            def _():
                pltpu.sync_copy(ones_v, cnt_sh.at[idx_v.at[j]], add=True)
        return carry

    lax.fori_loop(0, SCH, chunk, 0)
    plsc.subcore_barrier()

    pltpu.sync_copy(acc.at[pl.ds(rb, NRT)], s_out.at[c, pl.ds(rb, NRT)])
    if with_cnt:
        @pl.when(c == 0)
        def _():
            pltpu.sync_copy(cnt_sh.at[pl.ds(rb, NRT)], cnt_out.at[pl.ds(rb, NRT)])


@functools.cache
def _sc_mesh():
    return plsc.VectorSubcoreMesh(core_axis_name="c", subcore_axis_name="s")


@functools.cache
def _sc_gather_kernel(xw):
    return pl.kernel(
        _sc_gather_body,
        out_type=(jax.ShapeDtypeStruct((E_PAD, xw), jnp.float32),
                  jax.ShapeDtypeStruct((E_PAD, xw), jnp.float32)),
        mesh=_sc_mesh(),
        scratch_types=[pltpu.VMEM((GCH, KCH), jnp.int32),
                       pltpu.VMEM((GCH, KCH), jnp.int32),
                       pltpu.VMEM((KCH, xw), jnp.float32),
                       pltpu.VMEM((KCH, xw), jnp.float32),
                       pltpu.SemaphoreType.DMA,
                       pltpu.SemaphoreType.DMA],
    )


@functools.cache
def _sc_scatter_kernel(with_cnt):
    if with_cnt:
        return pl.kernel(
            functools.partial(_sc_scatter_body, True),
            out_type=(jax.ShapeDtypeStruct((NC, N_PAD, 128), jnp.float32),
                      jax.ShapeDtypeStruct((N_PAD,), jnp.float32)),
            mesh=_sc_mesh(),
            scratch_types=[pltpu.VMEM((SCH, KCH), jnp.int32),
                           pltpu.VMEM((KCH, 128), jnp.float32),
                           pltpu.VMEM((KCH,), jnp.float32),
                           pltpu.VMEM_SHARED((N_PAD, 128), jnp.float32),
                           pltpu.VMEM_SHARED((N_PAD,), jnp.float32)],
        )
    return pl.kernel(
        functools.partial(_sc_scatter_body, False),
        out_type=jax.ShapeDtypeStruct((NC, N_PAD, 128), jnp.float32),
        mesh=_sc_mesh(),
        scratch_types=[pltpu.VMEM((SCH, KCH), jnp.int32),
                       pltpu.VMEM((KCH, 128), jnp.float32),
                       pltpu.VMEM_SHARED((N_PAD, 128), jnp.float32)],
    )


def _gather_pallas(xtab, ridx3d, cidx3d):
    return _sc_gather_kernel(xtab.shape[1])(xtab, ridx3d, cidx3d)


def _scatter_pallas(h_st, cidx2d, zer2, zer1, with_cnt):
    if with_cnt:
        return _sc_scatter_kernel(True)(h_st, cidx2d, zer2, zer1)
    return _sc_scatter_kernel(False)(h_st, cidx2d, zer2), None


def kernel(x, edge_attr, params, edge_index, batch):
    f32 = jnp.float32
    row = edge_index[0].astype(jnp.int32)
    col = edge_index[1].astype(jnp.int32)
    padi = jnp.full((E_PAD - E_EDGES,), DUMMY, jnp.int32)
    ridx_g = jnp.concatenate([row, padi]).reshape(NW, GCH, KCH)
    cidx_full = jnp.concatenate([col, padi])
    cidx_g = cidx_full.reshape(NW, GCH, KCH)
    cidx_s = cidx_full.reshape(NS, SCH, KCH)

    xp = jnp.zeros((N_PAD, x.shape[1]), f32).at[:N_NODES].set(x)
    ep = jnp.zeros((E_PAD, edge_attr.shape[1]), f32).at[:E_EDGES].set(edge_attr)
    bpad = jnp.concatenate(
        [batch.astype(jnp.int32), jnp.full((N_PAD - N_NODES,), NGRAPH, jnp.int32)]
    ).reshape(N_PAD, 1)
    zer2 = jnp.zeros((N_PAD, 128), f32)
    zer1 = jnp.zeros((N_PAD,), f32)

    xs = _stats_x(x)
    es = _stats_e(edge_attr)
    xcur = _bn_apply(xp, xs, params["bn_node"]["g"], params["bn_node"]["b"], NBLK)
    e = _bn_apply(ep, es, params["bn_edge"]["g"], params["bn_edge"]["b"], NBLK)

    u = None
    cnt2d = None
    for li, mp in enumerate(params["metas"]):
        nf = xcur.shape[1]
        w1e, b1e = mp["edge"][0]["w"], mp["edge"][0]["b"]
        w2e, b2e = mp["edge"][1]["w"], mp["edge"][1]["b"]
        w1n, b1n = mp["node1"][0]["w"], mp["node1"][0]["b"]
        w2n, b2n = mp["node1"][1]["w"], mp["node1"][1]["b"]
        wxr, wxc, wee = w1e[:nf], w1e[nf:2 * nf], w1e[2 * nf:]
        wnx, whh = w1n[:nf], w1n[nf:]
        wrn = jnp.concatenate([wxr, wnx], axis=1)
        if nf % 128:
            nfp = 128
            xg = jnp.zeros((N_PAD, nfp), f32).at[:, :nf].set(xcur)
            wrn = jnp.zeros((nfp, wrn.shape[1]), f32).at[:nf].set(wrn)
            wxc = jnp.zeros((nfp, wxc.shape[1]), f32).at[:nf].set(wxc)
        else:
            xg = xcur

        g1, g2 = _gather_pallas(xg, ridx_g, cidx_g)
        e_new, h_st = _p2(e, g1, g2, wrn, wxc, wee, b1e.reshape(1, -1),
                          w2e, b2e.reshape(1, -1), whh, b1n.reshape(1, -1),
                          w2n, b2n.reshape(1, -1))
        S, cnt = _scatter_pallas(h_st, cidx_s, zer2, zer1, li == 0)
        if li == 0:
            cnt2d = cnt.reshape(N_PAD, 1)

        w1n2, b1n2 = mp["node2"][0]["w"], mp["node2"][0]["b"]
        w2n2, b2n2 = mp["node2"][1]["w"], mp["node2"][1]["b"]
        wx2, wh2 = w1n2[:nf], w1n2[nf:]
        wh2a, wh2b = wh2[:128], wh2[128:]
        w1g, b1g = mp["glob"][0]["w"], mp["glob"][0]["b"]
        w2g, b2g = mp["glob"][1]["w"], mp["glob"][1]["b"]
        ng = 0 if u is None else u.shape[1]
        wg1u = w1g[:ng] if ng else None
        wg1g = w1g[ng:]
        xcur, u = _p3(xcur, S, cnt2d, bpad, u, wx2, wh2a, wh2b,
                      b1n2.reshape(1, -1), w2n2, b2n2.reshape(1, -1),
                      wg1u, wg1g, b1g.reshape(1, -1), w2g, b2g.reshape(1, -1))
        e = e_new

    w1, b1 = params["lin1"]["w"], params["lin1"]["b"]
    w2, b2 = params["lin2"]["w"], params["lin2"]["b"]
    cout = w2.shape[1]
    cpad = 14592
    w2p = jnp.zeros((w2.shape[0], cpad), f32).at[:, :cout].set(w2)
    b2p = jnp.zeros((1, cpad), f32).at[0, :cout].set(b2)
    y = _p5(u, w1, b1.reshape(1, -1), w2p, b2p, 2432)
    return (u, y[:, :cout])


# speedup vs baseline: 1.0969x; 1.0969x over previous
import functools

import jax
import jax.numpy as jnp
from jax import lax
from jax.experimental import pallas as pl
from jax.experimental.pallas import tpu as pltpu
from jax.experimental.pallas import tpu_sc as plsc

N_NODES = 10000
N_PAD = 10240
E_EDGES = 100000
E_PAD = 102400
NGRAPH = 64
DUMMY = 10000
NBLK = 512
N_BLOCKS = N_PAD // NBLK
E_BLOCKS = E_PAD // NBLK
DR = 384

NC, NS = 2, 16
NW = NC * NS
KCH = 128
EW = E_PAD // NW
GCH = EW // KCH
ET = E_PAD // NS
SCH = ET // KCH
NRT = N_PAD // NS

_INTERPRET = False


def _elu(v):
    return jnp.where(v > 0, v, jnp.exp(jnp.minimum(v, 0.0)) - 1.0)


def _dot(a, b):
    return jnp.dot(a, b, preferred_element_type=jnp.float32)


def _stats_x_body(x_ref, o_ref):
    x = x_ref[...]
    m = jnp.mean(x, axis=0, keepdims=True)
    v = jnp.mean((x - m) ** 2, axis=0, keepdims=True)
    o_ref[...] = jnp.concatenate([m, v], axis=0)


def _stats_x(x):
    d = x.shape[1]
    return pl.pallas_call(
        _stats_x_body,
        out_shape=jax.ShapeDtypeStruct((2, d), jnp.float32),
        interpret=_INTERPRET,
    )(x)


def _stats_e_body(e_ref, o_ref, s_ref, q_ref):
    i = pl.program_id(0)
    e = e_ref[...]
    ps = jnp.sum(e, axis=0, keepdims=True)
    pq = jnp.sum(e * e, axis=0, keepdims=True)

    @pl.when(i == 0)
    def _():
        s_ref[...] = ps
        q_ref[...] = pq

    @pl.when(i > 0)
    def _():
        s_ref[...] += ps
        q_ref[...] += pq

    @pl.when(i == pl.num_programs(0) - 1)
    def _():
        m = s_ref[...] / E_EDGES
        v = q_ref[...] / E_EDGES - m * m
        o_ref[...] = jnp.concatenate([m, v], axis=0)


def _stats_e(e):
    d = e.shape[1]
    nb = e.shape[0] // 4000
    return pl.pallas_call(
        _stats_e_body,
        grid=(nb,),
        in_specs=[pl.BlockSpec((4000, d), lambda i: (i, 0))],
        out_specs=pl.BlockSpec((2, d), lambda i: (0, 0)),
        out_shape=jax.ShapeDtypeStruct((2, d), jnp.float32),
        scratch_shapes=[pltpu.VMEM((1, d), jnp.float32),
                        pltpu.VMEM((1, d), jnp.float32)],
        interpret=_INTERPRET,
    )(e)


def _bn_body(x_ref, st_ref, g_ref, b_ref, o_ref):
    st = st_ref[...]
    m, v = st[0:1], st[1:2]
    o_ref[...] = (x_ref[...] - m) / jnp.sqrt(v + 1e-5) * g_ref[...] + b_ref[...]


def _bn_apply(x, stats, g, b, blk):
    n, d = x.shape
    return pl.pallas_call(
        _bn_body,
        grid=(n // blk,),
        in_specs=[pl.BlockSpec((blk, d), lambda i: (i, 0)),
                  pl.BlockSpec((2, d), lambda i: (0, 0)),
                  pl.BlockSpec((1, d), lambda i: (0, 0)),
                  pl.BlockSpec((1, d), lambda i: (0, 0))],
        out_specs=pl.BlockSpec((blk, d), lambda i: (i, 0)),
        out_shape=jax.ShapeDtypeStruct((n, d), jnp.float32),
        interpret=_INTERPRET,
    )(x, stats, g.reshape(1, d), b.reshape(1, d))


def _p2_body(e_ref, g1_ref, g2_ref, wrn_ref, wc_ref,
             wee_ref, b1e_ref, w2e_ref, b2e_ref,
             wh_ref, b1n_ref, w2n_ref, b2n_ref, en_ref, h_ref):
    t1 = _dot(g1_ref[...], wrn_ref[...])
    h1 = _elu(t1[:, :128] + _dot(g2_ref[...], wc_ref[...])
              + _dot(e_ref[...], wee_ref[...]) + b1e_ref[...])
    e_new = _dot(h1, w2e_ref[...]) + b2e_ref[...]
    en_ref[...] = e_new
    n1 = _elu(t1[:, 128:] + _dot(e_new, wh_ref[...]) + b1n_ref[...])
    h = _dot(n1, w2n_ref[...]) + b2n_ref[...]
    h_ref[0, :, :] = h[:, :128]
    h_ref[1, :, :] = h[:, 128:]


def _p2(e, g1, g2, wrn, wc, wee, b1e, w2e, b2e, wh, b1n, w2n, b2n):
    ne = e.shape[1]
    xw = g1.shape[1]
    eo = w2e.shape[1]
    full = lambda a: pl.BlockSpec(a.shape, lambda i: tuple(0 for _ in a.shape))
    return pl.pallas_call(
        _p2_body,
        grid=(E_BLOCKS,),
        in_specs=[pl.BlockSpec((NBLK, ne), lambda i: (i, 0)),
                  pl.BlockSpec((NBLK, xw), lambda i: (i, 0)),
                  pl.BlockSpec((NBLK, xw), lambda i: (i, 0)),
                  full(wrn), full(wc),
                  full(wee), full(b1e), full(w2e), full(b2e),
                  full(wh), full(b1n), full(w2n), full(b2n)],
        out_specs=[pl.BlockSpec((NBLK, eo), lambda i: (i, 0)),
                   pl.BlockSpec((2, NBLK, 128), lambda i: (0, i, 0))],
        out_shape=[jax.ShapeDtypeStruct((E_PAD, eo), jnp.float32),
                   jax.ShapeDtypeStruct((2, E_PAD, 128), jnp.float32)],
        interpret=_INTERPRET,
    )(e, g1, g2, wrn, wc, wee, b1e, w2e, b2e, wh, b1n, w2n, b2n)


def _p3_body(has_u, no, go, *refs):
    if has_u:
        (xn_ref, s0_ref, s1_ref, cnt_ref, b_ref, u_ref,
         wx2_ref, wh2a_ref, wh2b_ref, b1_ref, w2_ref, b2_ref,
         wg1u_ref, wg1g_ref, bg1_ref, wg2_ref, bg2_ref,
         xo_ref, uo_ref, gm_ref, gc_ref) = refs
    else:
        (xn_ref, s0_ref, s1_ref, cnt_ref, b_ref,
         wx2_ref, wh2a_ref, wh2b_ref, b1_ref, w2_ref, b2_ref,
         wg1g_ref, bg1_ref, wg2_ref, bg2_ref,
         xo_ref, uo_ref, gm_ref, gc_ref) = refs
    i = pl.program_id(0)
    cnt = jnp.maximum(cnt_ref[...], 1.0)
    hm0 = s0_ref[0, :, :] / cnt
    hm1 = s1_ref[0, :, :] / cnt
    t = _elu(_dot(xn_ref[...], wx2_ref[...]) + _dot(hm0, wh2a_ref[...])
             + _dot(hm1, wh2b_ref[...]) + b1_ref[...])
    xnew = _dot(t, w2_ref[...]) + b2_ref[...]
    xo_ref[...] = xnew

    gids = lax.broadcasted_iota(jnp.int32, (1, NGRAPH), 1)
    oh = (b_ref[...] == gids).astype(jnp.float32)
    part = lax.dot_general(oh, xnew, (((0,), (0,)), ((), ())),
                           preferred_element_type=jnp.float32)
    ones = jnp.ones((NBLK, 1), jnp.float32)
    pcnt = lax.dot_general(oh, ones, (((0,), (0,)), ((), ())),
                           preferred_element_type=jnp.float32)

    @pl.when(i == 0)
    def _():
        gm_ref[...] = part
        gc_ref[...] = pcnt

    @pl.when(i > 0)
    def _():
        gm_ref[...] += part
        gc_ref[...] += pcnt

    @pl.when(i == pl.num_programs(0) - 1)
    def _():
        gmean = gm_ref[...] / jnp.maximum(gc_ref[...], 1.0)
        pre = _dot(gmean, wg1g_ref[...]) + bg1_ref[...]
        if has_u:
            pre = pre + _dot(u_ref[...], wg1u_ref[...])
        ug1 = _elu(pre)
        uo_ref[...] = _dot(ug1, wg2_ref[...]) + bg2_ref[...]


def _p3(xn, S, cnt2d, batch2d, u, wx2, wh2a, wh2b, b1, w2, b2,
        wg1u, wg1g, bg1, wg2, bg2):
    has_u = u is not None
    nf = xn.shape[1]
    no = w2.shape[1]
    go = wg2.shape[1]
    full = lambda a: pl.BlockSpec(a.shape, lambda i: tuple(0 for _ in a.shape))
    in_specs = [pl.BlockSpec((NBLK, nf), lambda i: (i, 0)),
                pl.BlockSpec((1, NBLK, 128), lambda i: (0, i, 0)),
                pl.BlockSpec((1, NBLK, 128), lambda i: (1, i, 0)),
                pl.BlockSpec((NBLK, 1), lambda i: (i, 0)),
                pl.BlockSpec((NBLK, 1), lambda i: (i, 0))]
    args = [xn, S, S, cnt2d, batch2d]
    if has_u:
        in_specs.append(full(u))
        args.append(u)
    wargs = [wx2, wh2a, wh2b, b1, w2, b2]
    if has_u:
        wargs.append(wg1u)
    wargs += [wg1g, bg1, wg2, bg2]
    in_specs += [full(a) for a in wargs]
    args += wargs
    body = functools.partial(_p3_body, has_u, no, go)
    return pl.pallas_call(
        body,
        grid=(N_BLOCKS,),
        in_specs=in_specs,
        out_specs=[pl.BlockSpec((NBLK, no), lambda i: (i, 0)),
                   pl.BlockSpec((NGRAPH, go), lambda i: (0, 0))],
        out_shape=[jax.ShapeDtypeStruct((N_PAD, no), jnp.float32),
                   jax.ShapeDtypeStruct((NGRAPH, go), jnp.float32)],
        scratch_shapes=[pltpu.VMEM((NGRAPH, no), jnp.float32),
                        pltpu.VMEM((NGRAPH, 1), jnp.float32)],
        interpret=_INTERPRET,
    )(*args)


def _p5_body(u_ref, w1_ref, b1_ref, w2_ref, b2_ref, o_ref):
    y1 = _elu(_dot(u_ref[...], w1_ref[...]) + b1_ref[...])
    o_ref[...] = _dot(y1, w2_ref[...]) + b2_ref[...]


def _p5(u, w1, b1, w2p, b2p, cblk):
    cpad = w2p.shape[1]
    full = lambda a: pl.BlockSpec(a.shape, lambda j: tuple(0 for _ in a.shape))
    return pl.pallas_call(
        _p5_body,
        grid=(cpad // cblk,),
        in_specs=[full(u), full(w1), full(b1),
                  pl.BlockSpec((w2p.shape[0], cblk), lambda j: (0, j)),
                  pl.BlockSpec((1, cblk), lambda j: (0, j))],
        out_specs=pl.BlockSpec((NGRAPH, cblk), lambda j: (0, j)),
        out_shape=jax.ShapeDtypeStruct((NGRAPH, cpad), jnp.float32),
        interpret=_INTERPRET,
    )(u, w1, b1, w2p, b2p)


def _sc_gather_body(xtab, ridx, cidx, g1, g2, ri_v, ci_v, b1_v, b2_v,
                    sr0, sr1, sc0, sc1):
    wid = lax.axis_index("s") * NC + lax.axis_index("c")
    pltpu.sync_copy(ridx.at[wid], ri_v)
    pltpu.sync_copy(cidx.at[wid], ci_v)
    srs = (sr0, sr1)
    scs = (sc0, sc1)

    def fire(j, p):
        return (pltpu.async_copy(xtab.at[ri_v.at[j]], b1_v.at[p], srs[p]),
                pltpu.async_copy(xtab.at[ci_v.at[j]], b2_v.at[p], scs[p]))

    cps = [fire(0, 0), None]
    for j in range(GCH):
        p = j % 2
        if j + 1 < GCH:
            cps[1 - p] = fire(j + 1, 1 - p)
        cps[p][0].wait()
        cps[p][1].wait()
        eb = wid * EW + j * KCH
        pltpu.sync_copy(b1_v.at[p], g1.at[pl.ds(eb, KCH)])
        pltpu.sync_copy(b2_v.at[p], g2.at[pl.ds(eb, KCH)])


def _sc_scatter_body(with_cnt, *refs):
    if with_cnt:
        (h_st, cidx, zer2, zer1, s_out, cnt_out,
         idx_v, h_v, ones_v, acc, cnt_sh, sh0, sh1) = refs
    else:
        (h_st, cidx, zer2, s_out, idx_v, h_v, acc, sh0, sh1) = refs
    c = lax.axis_index("c")
    s = lax.axis_index("s")
    rb = s * NRT
    shs = (sh0, sh1)
    pltpu.sync_copy(zer2.at[pl.ds(rb, NRT)], acc.at[pl.ds(rb, NRT)])
    if with_cnt:
        @pl.when(c == 0)
        def _():
            pltpu.sync_copy(zer1.at[pl.ds(rb, NRT)], cnt_sh.at[pl.ds(rb, NRT)])
        for l in range(KCH // 16):
            ones_v[pl.ds(l * 16, 16)] = jnp.ones((16,), jnp.float32)
    plsc.subcore_barrier()

    pltpu.sync_copy(cidx.at[s], idx_v)

    def fire(j, p):
        eb = s * ET + j * KCH
        pltpu.async_copy(h_st.at[c, pl.ds(eb, KCH)], h_v.at[p], shs[p])

    def consume(j, p):
        pltpu.make_async_copy(h_st.at[c, pl.ds(0, KCH)], h_v.at[p],
                              shs[p]).wait()
        pltpu.sync_copy(h_v.at[p], acc.at[idx_v.at[j]], add=True)
        if with_cnt:
            @pl.when(c == 0)
            def _():
                pltpu.sync_copy(ones_v, cnt_sh.at[idx_v.at[j]], add=True)

    fire(0, 0)
    fire(1, 1)

    @pl.loop(0, SCH - 2, step=2)
    def _(base):
        for b in range(2):
            consume(base + b, b)
            fire(base + b + 2, b)

    for b in range(2):
        consume(SCH - 2 + b, b)
    plsc.subcore_barrier()

    pltpu.sync_copy(acc.at[pl.ds(rb, NRT)], s_out.at[c, pl.ds(rb, NRT)])
    if with_cnt:
        @pl.when(c == 0)
        def _():
            pltpu.sync_copy(cnt_sh.at[pl.ds(rb, NRT)], cnt_out.at[pl.ds(rb, NRT)])


@functools.cache
def _sc_mesh():
    return plsc.VectorSubcoreMesh(core_axis_name="c", subcore_axis_name="s")


@functools.cache
def _sc_gather_kernel(xw):
    return pl.kernel(
        _sc_gather_body,
        out_type=(jax.ShapeDtypeStruct((E_PAD, xw), jnp.float32),
                  jax.ShapeDtypeStruct((E_PAD, xw), jnp.float32)),
        mesh=_sc_mesh(),
        scratch_types=[pltpu.VMEM((GCH, KCH), jnp.int32),
                       pltpu.VMEM((GCH, KCH), jnp.int32),
                       pltpu.VMEM((2, KCH, xw), jnp.float32),
                       pltpu.VMEM((2, KCH, xw), jnp.float32),
                       pltpu.SemaphoreType.DMA,
                       pltpu.SemaphoreType.DMA,
                       pltpu.SemaphoreType.DMA,
                       pltpu.SemaphoreType.DMA],
    )


@functools.cache
def _sc_scatter_kernel(with_cnt):
    if with_cnt:
        return pl.kernel(
            functools.partial(_sc_scatter_body, True),
            out_type=(jax.ShapeDtypeStruct((NC, N_PAD, 128), jnp.float32),
                      jax.ShapeDtypeStruct((N_PAD,), jnp.float32)),
            mesh=_sc_mesh(),
            scratch_types=[pltpu.VMEM((SCH, KCH), jnp.int32),
                           pltpu.VMEM((2, KCH, 128), jnp.float32),
                           pltpu.VMEM((KCH,), jnp.float32),
                           pltpu.VMEM_SHARED((N_PAD, 128), jnp.float32),
                           pltpu.VMEM_SHARED((N_PAD,), jnp.float32),
                           pltpu.SemaphoreType.DMA,
                           pltpu.SemaphoreType.DMA],
        )
    return pl.kernel(
        functools.partial(_sc_scatter_body, False),
        out_type=jax.ShapeDtypeStruct((NC, N_PAD, 128), jnp.float32),
        mesh=_sc_mesh(),
        scratch_types=[pltpu.VMEM((SCH, KCH), jnp.int32),
                       pltpu.VMEM((2, KCH, 128), jnp.float32),
                       pltpu.VMEM_SHARED((N_PAD, 128), jnp.float32),
                       pltpu.SemaphoreType.DMA,
                       pltpu.SemaphoreType.DMA],
    )


def _gather_pallas(xtab, ridx3d, cidx3d):
    return _sc_gather_kernel(xtab.shape[1])(xtab, ridx3d, cidx3d)


def _scatter_pallas(h_st, cidx2d, zer2, zer1, with_cnt):
    if with_cnt:
        return _sc_scatter_kernel(True)(h_st, cidx2d, zer2, zer1)
    return _sc_scatter_kernel(False)(h_st, cidx2d, zer2), None


def kernel(x, edge_attr, params, edge_index, batch):
    f32 = jnp.float32
    row = edge_index[0].astype(jnp.int32)
    col = edge_index[1].astype(jnp.int32)
    padi = jnp.full((E_PAD - E_EDGES,), DUMMY, jnp.int32)
    ridx_g = jnp.concatenate([row, padi]).reshape(NW, GCH, KCH)
    cidx_full = jnp.concatenate([col, padi])
    cidx_g = cidx_full.reshape(NW, GCH, KCH)
    cidx_s = cidx_full.reshape(NS, SCH, KCH)

    xp = jnp.zeros((N_PAD, x.shape[1]), f32).at[:N_NODES].set(x)
    ep = jnp.zeros((E_PAD, edge_attr.shape[1]), f32).at[:E_EDGES].set(edge_attr)
    bpad = jnp.concatenate(
        [batch.astype(jnp.int32), jnp.full((N_PAD - N_NODES,), NGRAPH, jnp.int32)]
    ).reshape(N_PAD, 1)
    zer2 = jnp.zeros((N_PAD, 128), f32)
    zer1 = jnp.zeros((N_PAD,), f32)

    xs = _stats_x(x)
    es = _stats_e(edge_attr)
    xcur = _bn_apply(xp, xs, params["bn_node"]["g"], params["bn_node"]["b"], NBLK)
    e = _bn_apply(ep, es, params["bn_edge"]["g"], params["bn_edge"]["b"], NBLK)

    u = None
    cnt2d = None
    for li, mp in enumerate(params["metas"]):
        nf = xcur.shape[1]
        w1e, b1e = mp["edge"][0]["w"], mp["edge"][0]["b"]
        w2e, b2e = mp["edge"][1]["w"], mp["edge"][1]["b"]
        w1n, b1n = mp["node1"][0]["w"], mp["node1"][0]["b"]
        w2n, b2n = mp["node1"][1]["w"], mp["node1"][1]["b"]
        wxr, wxc, wee = w1e[:nf], w1e[nf:2 * nf], w1e[2 * nf:]
        wnx, whh = w1n[:nf], w1n[nf:]
        wrn = jnp.concatenate([wxr, wnx], axis=1)
        if nf % 128:
            nfp = 128
            xg = jnp.zeros((N_PAD, nfp), f32).at[:, :nf].set(xcur)
            wrn = jnp.zeros((nfp, wrn.shape[1]), f32).at[:nf].set(wrn)
            wxc = jnp.zeros((nfp, wxc.shape[1]), f32).at[:nf].set(wxc)
        else:
            xg = xcur

        g1, g2 = _gather_pallas(xg, ridx_g, cidx_g)
        e_new, h_st = _p2(e, g1, g2, wrn, wxc, wee, b1e.reshape(1, -1),
                          w2e, b2e.reshape(1, -1), whh, b1n.reshape(1, -1),
                          w2n, b2n.reshape(1, -1))
        S, cnt = _scatter_pallas(h_st, cidx_s, zer2, zer1, li == 0)
        if li == 0:
            cnt2d = cnt.reshape(N_PAD, 1)

        w1n2, b1n2 = mp["node2"][0]["w"], mp["node2"][0]["b"]
        w2n2, b2n2 = mp["node2"][1]["w"], mp["node2"][1]["b"]
        wx2, wh2 = w1n2[:nf], w1n2[nf:]
        wh2a, wh2b = wh2[:128], wh2[128:]
        w1g, b1g = mp["glob"][0]["w"], mp["glob"][0]["b"]
        w2g, b2g = mp["glob"][1]["w"], mp["glob"][1]["b"]
        ng = 0 if u is None else u.shape[1]
        wg1u = w1g[:ng] if ng else None
        wg1g = w1g[ng:]
        xcur, u = _p3(xcur, S, cnt2d, bpad, u, wx2, wh2a, wh2b,
                      b1n2.reshape(1, -1), w2n2, b2n2.reshape(1, -1),
                      wg1u, wg1g, b1g.reshape(1, -1), w2g, b2g.reshape(1, -1))
        e = e_new

    w1, b1 = params["lin1"]["w"], params["lin1"]["b"]
    w2, b2 = params["lin2"]["w"], params["lin2"]["b"]
    cout = w2.shape[1]
    cpad = 14592
    w2p = jnp.zeros((w2.shape[0], cpad), f32).at[:, :cout].set(w2)
    b2p = jnp.zeros((1, cpad), f32).at[0, :cout].set(b2)
    y = _p5(u, w1, b1.reshape(1, -1), w2p, b2p, 2432)
    return (u, y[:, :cout])


# speedup vs baseline: 1.1521x; 1.0504x over previous
import functools

import jax
import jax.numpy as jnp
from jax import lax
from jax.experimental import pallas as pl
from jax.experimental.pallas import tpu as pltpu
from jax.experimental.pallas import tpu_sc as plsc

N_NODES = 10000
N_PAD = 10240
E_EDGES = 100000
E_PAD = 102400
NGRAPH = 64
DUMMY = 10000
NBLK = 512
N_BLOCKS = N_PAD // NBLK
E_BLOCKS = E_PAD // NBLK
DR = 384

NC, NS = 2, 16
NW = NC * NS
KCH = 128
EW = E_PAD // NW
GCH = EW // KCH
ET = E_PAD // NS
SCH = ET // KCH
NRT = N_PAD // NS

_INTERPRET = False


def _elu(v):
    return jnp.where(v > 0, v, jnp.exp(jnp.minimum(v, 0.0)) - 1.0)


def _dot(a, b):
    return jnp.dot(a, b, preferred_element_type=jnp.float32)


def _stats_x_body(x_ref, o_ref):
    x = x_ref[...]
    m = jnp.mean(x, axis=0, keepdims=True)
    v = jnp.mean((x - m) ** 2, axis=0, keepdims=True)
    o_ref[...] = jnp.concatenate([m, v], axis=0)


def _stats_x(x):
    d = x.shape[1]
    return pl.pallas_call(
        _stats_x_body,
        out_shape=jax.ShapeDtypeStruct((2, d), jnp.float32),
        interpret=_INTERPRET,
    )(x)


def _stats_e_body(e_ref, o_ref, s_ref, q_ref):
    i = pl.program_id(0)
    e = e_ref[...]
    ps = jnp.sum(e, axis=0, keepdims=True)
    pq = jnp.sum(e * e, axis=0, keepdims=True)

    @pl.when(i == 0)
    def _():
        s_ref[...] = ps
        q_ref[...] = pq

    @pl.when(i > 0)
    def _():
        s_ref[...] += ps
        q_ref[...] += pq

    @pl.when(i == pl.num_programs(0) - 1)
    def _():
        m = s_ref[...] / E_EDGES
        v = q_ref[...] / E_EDGES - m * m
        o_ref[...] = jnp.concatenate([m, v], axis=0)


def _stats_e(e):
    d = e.shape[1]
    nb = e.shape[0] // 4000
    return pl.pallas_call(
        _stats_e_body,
        grid=(nb,),
        in_specs=[pl.BlockSpec((4000, d), lambda i: (i, 0))],
        out_specs=pl.BlockSpec((2, d), lambda i: (0, 0)),
        out_shape=jax.ShapeDtypeStruct((2, d), jnp.float32),
        scratch_shapes=[pltpu.VMEM((1, d), jnp.float32),
                        pltpu.VMEM((1, d), jnp.float32)],
        interpret=_INTERPRET,
    )(e)


def _bn_body(x_ref, st_ref, g_ref, b_ref, o_ref):
    st = st_ref[...]
    m, v = st[0:1], st[1:2]
    o_ref[...] = (x_ref[...] - m) / jnp.sqrt(v + 1e-5) * g_ref[...] + b_ref[...]


def _bn_apply(x, stats, g, b, blk):
    n, d = x.shape
    return pl.pallas_call(
        _bn_body,
        grid=(n // blk,),
        in_specs=[pl.BlockSpec((blk, d), lambda i: (i, 0)),
                  pl.BlockSpec((2, d), lambda i: (0, 0)),
                  pl.BlockSpec((1, d), lambda i: (0, 0)),
                  pl.BlockSpec((1, d), lambda i: (0, 0))],
        out_specs=pl.BlockSpec((blk, d), lambda i: (i, 0)),
        out_shape=jax.ShapeDtypeStruct((n, d), jnp.float32),
        interpret=_INTERPRET,
    )(x, stats, g.reshape(1, d), b.reshape(1, d))


def _p1_body(x_ref, wr_ref, wc_ref, tr_ref, tc_ref):
    x = x_ref[...]
    tr_ref[...] = _dot(x, wr_ref[...])
    tc_ref[...] = _dot(x, wc_ref[...])


def _p1(xn, w_row, w_col):
    nf = xn.shape[1]
    return pl.pallas_call(
        _p1_body,
        grid=(N_BLOCKS,),
        in_specs=[pl.BlockSpec((NBLK, nf), lambda i: (i, 0)),
                  pl.BlockSpec((nf, DR), lambda i: (0, 0)),
                  pl.BlockSpec((nf, 128), lambda i: (0, 0))],
        out_specs=[pl.BlockSpec((NBLK, DR), lambda i: (i, 0)),
                   pl.BlockSpec((NBLK, 128), lambda i: (i, 0))],
        out_shape=[jax.ShapeDtypeStruct((N_PAD, DR), jnp.float32),
                   jax.ShapeDtypeStruct((N_PAD, 128), jnp.float32)],
        interpret=_INTERPRET,
    )(xn, w_row, w_col)


def _fold_body(a_ref, b_ref, ba_ref, bb_ref, w_ref, bo_ref):
    w_ref[...] = _dot(a_ref[...], b_ref[...])
    bo_ref[...] = _dot(ba_ref[...], b_ref[...]) + bb_ref[...]


def _fold(a, b, ba, bb):
    full = lambda s: pl.BlockSpec(s.shape, lambda: tuple(0 for _ in s.shape))
    return pl.pallas_call(
        _fold_body,
        in_specs=[full(a), full(b), full(ba), full(bb)],
        out_specs=[pl.BlockSpec((a.shape[0], b.shape[1]), lambda: (0, 0)),
                   pl.BlockSpec((1, b.shape[1]), lambda: (0, 0))],
        out_shape=[jax.ShapeDtypeStruct((a.shape[0], b.shape[1]), jnp.float32),
                   jax.ShapeDtypeStruct((1, b.shape[1]), jnp.float32)],
        interpret=_INTERPRET,
    )(a, b, ba, bb)


def _p2_body(hp_ref, g1_ref, g2_ref, wef_ref, bef_ref, weh_ref, beh_ref,
             h1_ref, n1_ref):
    h1 = _elu(g1_ref[:, :128] + g2_ref[...]
              + _dot(hp_ref[...], wef_ref[...]) + bef_ref[...])
    h1_ref[...] = h1
    n1 = _elu(g1_ref[:, 128:] + _dot(h1, weh_ref[...]) + beh_ref[...])
    n1_ref[0, :, :] = n1[:, :128]
    n1_ref[1, :, :] = n1[:, 128:]


def _p2(hprev, g1, g2, wef, bef, weh, beh):
    ne = hprev.shape[1]
    full = lambda a: pl.BlockSpec(a.shape, lambda i: tuple(0 for _ in a.shape))
    return pl.pallas_call(
        _p2_body,
        grid=(E_BLOCKS,),
        in_specs=[pl.BlockSpec((NBLK, ne), lambda i: (i, 0)),
                  pl.BlockSpec((NBLK, DR), lambda i: (i, 0)),
                  pl.BlockSpec((NBLK, 128), lambda i: (i, 0)),
                  full(wef), full(bef), full(weh), full(beh)],
        out_specs=[pl.BlockSpec((NBLK, 128), lambda i: (i, 0)),
                   pl.BlockSpec((2, NBLK, 128), lambda i: (0, i, 0))],
        out_shape=[jax.ShapeDtypeStruct((E_PAD, 128), jnp.float32),
                   jax.ShapeDtypeStruct((2, E_PAD, 128), jnp.float32)],
        interpret=_INTERPRET,
    )(hprev, g1, g2, wef, bef, weh, beh)


def _p3_body(has_u, no, go, *refs):
    if has_u:
        (xn_ref, s0_ref, s1_ref, cnt_ref, b_ref, u_ref,
         wx2_ref, wh2a_ref, wh2b_ref, bv_ref, b1_ref, w2_ref, b2_ref,
         wg1u_ref, wg1g_ref, bg1_ref, wg2_ref, bg2_ref,
         xo_ref, uo_ref, gm_ref, gc_ref) = refs
    else:
        (xn_ref, s0_ref, s1_ref, cnt_ref, b_ref,
         wx2_ref, wh2a_ref, wh2b_ref, bv_ref, b1_ref, w2_ref, b2_ref,
         wg1g_ref, bg1_ref, wg2_ref, bg2_ref,
         xo_ref, uo_ref, gm_ref, gc_ref) = refs
    i = pl.program_id(0)
    cnt_raw = cnt_ref[...]
    cnt = jnp.maximum(cnt_raw, 1.0)
    he = (cnt_raw > 0).astype(jnp.float32)
    hm0 = s0_ref[0, :, :] / cnt
    hm1 = s1_ref[0, :, :] / cnt
    t = _elu(_dot(xn_ref[...], wx2_ref[...]) + _dot(hm0, wh2a_ref[...])
             + _dot(hm1, wh2b_ref[...]) + he * bv_ref[...] + b1_ref[...])
    xnew = _dot(t, w2_ref[...]) + b2_ref[...]
    xo_ref[...] = xnew

    gids = lax.broadcasted_iota(jnp.int32, (1, NGRAPH), 1)
    oh = (b_ref[...] == gids).astype(jnp.float32)
    part = lax.dot_general(oh, xnew, (((0,), (0,)), ((), ())),
                           preferred_element_type=jnp.float32)
    ones = jnp.ones((NBLK, 1), jnp.float32)
    pcnt = lax.dot_general(oh, ones, (((0,), (0,)), ((), ())),
                           preferred_element_type=jnp.float32)

    @pl.when(i == 0)
    def _():
        gm_ref[...] = part
        gc_ref[...] = pcnt

    @pl.when(i > 0)
    def _():
        gm_ref[...] += part
        gc_ref[...] += pcnt

    @pl.when(i == pl.num_programs(0) - 1)
    def _():
        gmean = gm_ref[...] / jnp.maximum(gc_ref[...], 1.0)
        pre = _dot(gmean, wg1g_ref[...]) + bg1_ref[...]
        if has_u:
            pre = pre + _dot(u_ref[...], wg1u_ref[...])
        ug1 = _elu(pre)
        uo_ref[...] = _dot(ug1, wg2_ref[...]) + bg2_ref[...]


def _p3(xn, S, cnt2d, batch2d, u, wx2, wh2a, wh2b, bv, b1, w2, b2,
        wg1u, wg1g, bg1, wg2, bg2):
    has_u = u is not None
    nf = xn.shape[1]
    no = w2.shape[1]
    go = wg2.shape[1]
    full = lambda a: pl.BlockSpec(a.shape, lambda i: tuple(0 for _ in a.shape))
    in_specs = [pl.BlockSpec((NBLK, nf), lambda i: (i, 0)),
                pl.BlockSpec((1, NBLK, 128), lambda i: (0, i, 0)),
                pl.BlockSpec((1, NBLK, 128), lambda i: (1, i, 0)),
                pl.BlockSpec((NBLK, 1), lambda i: (i, 0)),
                pl.BlockSpec((NBLK, 1), lambda i: (i, 0))]
    args = [xn, S, S, cnt2d, batch2d]
    if has_u:
        in_specs.append(full(u))
        args.append(u)
    wargs = [wx2, wh2a, wh2b, bv, b1, w2, b2]
    if has_u:
        wargs.append(wg1u)
    wargs += [wg1g, bg1, wg2, bg2]
    in_specs += [full(a) for a in wargs]
    args += wargs
    body = functools.partial(_p3_body, has_u, no, go)
    return pl.pallas_call(
        body,
        grid=(N_BLOCKS,),
        in_specs=in_specs,
        out_specs=[pl.BlockSpec((NBLK, no), lambda i: (i, 0)),
                   pl.BlockSpec((NGRAPH, go), lambda i: (0, 0))],
        out_shape=[jax.ShapeDtypeStruct((N_PAD, no), jnp.float32),
                   jax.ShapeDtypeStruct((NGRAPH, go), jnp.float32)],
        scratch_shapes=[pltpu.VMEM((NGRAPH, no), jnp.float32),
                        pltpu.VMEM((NGRAPH, 1), jnp.float32)],
        interpret=_INTERPRET,
    )(*args)


def _p5_body(u_ref, w1_ref, b1_ref, w2_ref, b2_ref, o_ref):
    y1 = _elu(_dot(u_ref[...], w1_ref[...]) + b1_ref[...])
    o_ref[...] = _dot(y1, w2_ref[...]) + b2_ref[...]


def _p5(u, w1, b1, w2p, b2p, cblk):
    cpad = w2p.shape[1]
    full = lambda a: pl.BlockSpec(a.shape, lambda j: tuple(0 for _ in a.shape))
    return pl.pallas_call(
        _p5_body,
        grid=(cpad // cblk,),
        in_specs=[full(u), full(w1), full(b1),
                  pl.BlockSpec((w2p.shape[0], cblk), lambda j: (0, j)),
                  pl.BlockSpec((1, cblk), lambda j: (0, j))],
        out_specs=pl.BlockSpec((NGRAPH, cblk), lambda j: (0, j)),
        out_shape=jax.ShapeDtypeStruct((NGRAPH, cpad), jnp.float32),
        interpret=_INTERPRET,
    )(u, w1, b1, w2p, b2p)


def _sc_gather_body(trow, tcol, ridx, cidx, g1, g2, ri_v, ci_v, b1_v, b2_v,
                    sr0, sr1, sc0, sc1):
    wid = lax.axis_index("s") * NC + lax.axis_index("c")
    pltpu.sync_copy(ridx.at[wid], ri_v)
    pltpu.sync_copy(cidx.at[wid], ci_v)
    srs = (sr0, sr1)
    del sc1

    def fire_row(j, p):
        return pltpu.async_copy(trow.at[ri_v.at[j]], b1_v.at[p], srs[p])

    def fire_col(j):
        return pltpu.async_copy(tcol.at[ci_v.at[j]], b2_v, sc0)

    cps = [fire_row(0, 0), None]
    cpc = fire_col(0)
    for j in range(GCH):
        p = j % 2
        if j + 1 < GCH:
            cps[1 - p] = fire_row(j + 1, 1 - p)
        cps[p].wait()
        cpc.wait()
        eb = wid * EW + j * KCH
        pltpu.sync_copy(b1_v.at[p], g1.at[pl.ds(eb, KCH)])
        pltpu.sync_copy(b2_v, g2.at[pl.ds(eb, KCH)])
        if j + 1 < GCH:
            cpc = fire_col(j + 1)


def _sc_scatter_body(with_cnt, *refs):
    if with_cnt:
        (h_st, cidx, zer2, zer1, s_out, cnt_out,
         idx_v, h_v, ones_v, acc, cnt_sh, sh0, sh1) = refs
    else:
        (h_st, cidx, zer2, s_out, idx_v, h_v, acc, sh0, sh1) = refs
    c = lax.axis_index("c")
    s = lax.axis_index("s")
    rb = s * NRT
    shs = (sh0, sh1)
    pltpu.sync_copy(zer2.at[pl.ds(rb, NRT)], acc.at[pl.ds(rb, NRT)])
    if with_cnt:
        @pl.when(c == 0)
        def _():
            pltpu.sync_copy(zer1.at[pl.ds(rb, NRT)], cnt_sh.at[pl.ds(rb, NRT)])
        for l in range(KCH // 16):
            ones_v[pl.ds(l * 16, 16)] = jnp.ones((16,), jnp.float32)
    plsc.subcore_barrier()

    pltpu.sync_copy(cidx.at[s], idx_v)

    def fire(j, p):
        eb = s * ET + j * KCH
        pltpu.async_copy(h_st.at[c, pl.ds(eb, KCH)], h_v.at[p], shs[p])

    def consume(j, p):
        pltpu.make_async_copy(h_st.at[c, pl.ds(0, KCH)], h_v.at[p],
                              shs[p]).wait()
        pltpu.sync_copy(h_v.at[p], acc.at[idx_v.at[j]], add=True)
        if with_cnt:
            @pl.when(c == 0)
            def _():
                pltpu.sync_copy(ones_v, cnt_sh.at[idx_v.at[j]], add=True)

    fire(0, 0)
    fire(1, 1)

    @pl.loop(0, SCH - 2, step=2)
    def _(base):
        for b in range(2):
            consume(base + b, b)
            fire(base + b + 2, b)

    for b in range(2):
        consume(SCH - 2 + b, b)
    plsc.subcore_barrier()

    pltpu.sync_copy(acc.at[pl.ds(rb, NRT)], s_out.at[c, pl.ds(rb, NRT)])
    if with_cnt:
        @pl.when(c == 0)
        def _():
            pltpu.sync_copy(cnt_sh.at[pl.ds(rb, NRT)], cnt_out.at[pl.ds(rb, NRT)])


@functools.cache
def _sc_mesh():
    return plsc.VectorSubcoreMesh(core_axis_name="c", subcore_axis_name="s")


@functools.cache
def _sc_gather_kernel():
    return pl.kernel(
        _sc_gather_body,
        out_type=(jax.ShapeDtypeStruct((E_PAD, DR), jnp.float32),
                  jax.ShapeDtypeStruct((E_PAD, 128), jnp.float32)),
        mesh=_sc_mesh(),
        scratch_types=[pltpu.VMEM((GCH, KCH), jnp.int32),
                       pltpu.VMEM((GCH, KCH), jnp.int32),
                       pltpu.VMEM((2, KCH, DR), jnp.float32),
                       pltpu.VMEM((KCH, 128), jnp.float32),
                       pltpu.SemaphoreType.DMA,
                       pltpu.SemaphoreType.DMA,
                       pltpu.SemaphoreType.DMA,
                       pltpu.SemaphoreType.DMA],
    )


@functools.cache
def _sc_scatter_kernel(with_cnt):
    if with_cnt:
        return pl.kernel(
            functools.partial(_sc_scatter_body, True),
            out_type=(jax.ShapeDtypeStruct((NC, N_PAD, 128), jnp.float32),
                      jax.ShapeDtypeStruct((N_PAD,), jnp.float32)),
            mesh=_sc_mesh(),
            scratch_types=[pltpu.VMEM((SCH, KCH), jnp.int32),
                           pltpu.VMEM((2, KCH, 128), jnp.float32),
                           pltpu.VMEM((KCH,), jnp.float32),
                           pltpu.VMEM_SHARED((N_PAD, 128), jnp.float32),
                           pltpu.VMEM_SHARED((N_PAD,), jnp.float32),
                           pltpu.SemaphoreType.DMA,
                           pltpu.SemaphoreType.DMA],
        )
    return pl.kernel(
        functools.partial(_sc_scatter_body, False),
        out_type=jax.ShapeDtypeStruct((NC, N_PAD, 128), jnp.float32),
        mesh=_sc_mesh(),
        scratch_types=[pltpu.VMEM((SCH, KCH), jnp.int32),
                       pltpu.VMEM((2, KCH, 128), jnp.float32),
                       pltpu.VMEM_SHARED((N_PAD, 128), jnp.float32),
                       pltpu.SemaphoreType.DMA,
                       pltpu.SemaphoreType.DMA],
    )


def _gather_pallas(trow, tcol, ridx3d, cidx3d):
    return _sc_gather_kernel()(trow, tcol, ridx3d, cidx3d)


def _scatter_pallas(h_st, cidx2d, zer2, zer1, with_cnt):
    if with_cnt:
        return _sc_scatter_kernel(True)(h_st, cidx2d, zer2, zer1)
    return _sc_scatter_kernel(False)(h_st, cidx2d, zer2), None


def kernel(x, edge_attr, params, edge_index, batch):
    f32 = jnp.float32
    row = edge_index[0].astype(jnp.int32)
    col = edge_index[1].astype(jnp.int32)
    padi = jnp.full((E_PAD - E_EDGES,), DUMMY, jnp.int32)
    ridx_g = jnp.concatenate([row, padi]).reshape(NW, GCH, KCH)
    cidx_full = jnp.concatenate([col, padi])
    cidx_g = cidx_full.reshape(NW, GCH, KCH)
    cidx_s = cidx_full.reshape(NS, SCH, KCH)

    xp = jnp.zeros((N_PAD, x.shape[1]), f32).at[:N_NODES].set(x)
    ep = jnp.zeros((E_PAD, edge_attr.shape[1]), f32).at[:E_EDGES].set(edge_attr)
    bpad = jnp.concatenate(
        [batch.astype(jnp.int32), jnp.full((N_PAD - N_NODES,), NGRAPH, jnp.int32)]
    ).reshape(N_PAD, 1)
    zer2 = jnp.zeros((N_PAD, 128), f32)
    zer1 = jnp.zeros((N_PAD,), f32)

    xs = _stats_x(x)
    es = _stats_e(edge_attr)
    xcur = _bn_apply(xp, xs, params["bn_node"]["g"], params["bn_node"]["b"], NBLK)
    e = _bn_apply(ep, es, params["bn_edge"]["g"], params["bn_edge"]["b"], NBLK)

    u = None
    cnt2d = None
    hprev = e
    w2e_prev = b2e_prev = None
    for li, mp in enumerate(params["metas"]):
        nf = xcur.shape[1]
        w1e, b1e = mp["edge"][0]["w"], mp["edge"][0]["b"]
        w2e, b2e = mp["edge"][1]["w"], mp["edge"][1]["b"]
        w1n, b1n = mp["node1"][0]["w"], mp["node1"][0]["b"]
        w2n1, b2n1 = mp["node1"][1]["w"], mp["node1"][1]["b"]
        wxr, wxc, wee = w1e[:nf], w1e[nf:2 * nf], w1e[2 * nf:]
        wnx, whh = w1n[:nf], w1n[nf:]
        wrn = jnp.concatenate([wxr, wnx], axis=1)

        trow, tcol = _p1(xcur, wrn, wxc)
        g1, g2 = _gather_pallas(trow, tcol, ridx_g, cidx_g)

        if li == 0:
            wef, bef = wee, b1e.reshape(1, -1)
        else:
            wef, bef = _fold(w2e_prev, wee, b2e_prev.reshape(1, -1),
                             b1e.reshape(1, -1))
        weh, beh = _fold(w2e, whh, b2e.reshape(1, -1), b1n.reshape(1, -1))
        h1, n1_st = _p2(hprev, g1, g2, wef, bef, weh, beh)
        S, cnt = _scatter_pallas(n1_st, cidx_s, zer2, zer1, li == 0)
        if li == 0:
            cnt2d = cnt.reshape(N_PAD, 1)

        w1n2, b1n2 = mp["node2"][0]["w"], mp["node2"][0]["b"]
        w2n2, b2n2 = mp["node2"][1]["w"], mp["node2"][1]["b"]
        wx2, wh2 = w1n2[:nf], w1n2[nf:]
        wf, bv = _fold(w2n1, wh2, b2n1.reshape(1, -1),
                       jnp.zeros((1, wh2.shape[1]), f32))
        wh2a, wh2b = wf[:128], wf[128:]
        w1g, b1g = mp["glob"][0]["w"], mp["glob"][0]["b"]
        w2g, b2g = mp["glob"][1]["w"], mp["glob"][1]["b"]
        ng = 0 if u is None else u.shape[1]
        wg1u = w1g[:ng] if ng else None
        wg1g = w1g[ng:]
        xcur, u = _p3(xcur, S, cnt2d, bpad, u, wx2, wh2a, wh2b, bv,
                      b1n2.reshape(1, -1), w2n2, b2n2.reshape(1, -1),
                      wg1u, wg1g, b1g.reshape(1, -1), w2g, b2g.reshape(1, -1))
        hprev = h1
        w2e_prev, b2e_prev = w2e, b2e

    w1, b1 = params["lin1"]["w"], params["lin1"]["b"]
    w2, b2 = params["lin2"]["w"], params["lin2"]["b"]
    cout = w2.shape[1]
    cpad = 14592
    w2p = jnp.zeros((w2.shape[0], cpad), f32).at[:, :cout].set(w2)
    b2p = jnp.zeros((1, cpad), f32).at[0, :cout].set(b2)
    y = _p5(u, w1, b1.reshape(1, -1), w2p, b2p, 2432)
    return (u, y[:, :cout])


# speedup vs baseline: 2.0034x; 1.7389x over previous
import functools

import jax
import jax.numpy as jnp
from jax import lax
from jax.experimental import pallas as pl
from jax.experimental.pallas import tpu as pltpu
from jax.experimental.pallas import tpu_sc as plsc

N_NODES = 10000
N_PAD = 10240
E_EDGES = 100000
E_PAD = 102400
NGRAPH = 64
DUMMY = 10000
NBLK = 512
N_BLOCKS = N_PAD // NBLK
E_BLOCKS = E_PAD // NBLK
DR = 384
HDR = DR // 2

NC, NS = 2, 16
NW = NC * NS
KCH = 128
EW = E_PAD // NW
GCH = EW // KCH
ET = E_PAD // NS
SCH = ET // KCH
NRT = N_PAD // NS

_INTERPRET = False


def _elu(v):
    return jnp.where(v > 0, v, jnp.exp(jnp.minimum(v, 0.0)) - 1.0)


def _dot(a, b):
    return jnp.dot(a, b, preferred_element_type=jnp.float32)


def _stats_x_body(x_ref, o_ref):
    x = x_ref[...]
    m = jnp.mean(x, axis=0, keepdims=True)
    v = jnp.mean((x - m) ** 2, axis=0, keepdims=True)
    o_ref[...] = jnp.concatenate([m, v], axis=0)


def _stats_x(x):
    d = x.shape[1]
    return pl.pallas_call(
        _stats_x_body,
        out_shape=jax.ShapeDtypeStruct((2, d), jnp.float32),
        interpret=_INTERPRET,
    )(x)


def _stats_e_body(e_ref, o_ref, s_ref, q_ref):
    i = pl.program_id(0)
    e = e_ref[...]
    ps = jnp.sum(e, axis=0, keepdims=True)
    pq = jnp.sum(e * e, axis=0, keepdims=True)

    @pl.when(i == 0)
    def _():
        s_ref[...] = ps
        q_ref[...] = pq

    @pl.when(i > 0)
    def _():
        s_ref[...] += ps
        q_ref[...] += pq

    @pl.when(i == pl.num_programs(0) - 1)
    def _():
        m = s_ref[...] / E_EDGES
        v = q_ref[...] / E_EDGES - m * m
        o_ref[...] = jnp.concatenate([m, v], axis=0)


def _stats_e(e):
    d = e.shape[1]
    nb = e.shape[0] // 4000
    return pl.pallas_call(
        _stats_e_body,
        grid=(nb,),
        in_specs=[pl.BlockSpec((4000, d), lambda i: (i, 0))],
        out_specs=pl.BlockSpec((2, d), lambda i: (0, 0)),
        out_shape=jax.ShapeDtypeStruct((2, d), jnp.float32),
        scratch_shapes=[pltpu.VMEM((1, d), jnp.float32),
                        pltpu.VMEM((1, d), jnp.float32)],
        interpret=_INTERPRET,
    )(e)


def _bn_body(x_ref, st_ref, g_ref, b_ref, o_ref):
    st = st_ref[...]
    m, v = st[0:1], st[1:2]
    o_ref[...] = (x_ref[...] - m) / jnp.sqrt(v + 1e-5) * g_ref[...] + b_ref[...]


def _bn_apply(x, stats, g, b, blk):
    n, d = x.shape
    return pl.pallas_call(
        _bn_body,
        grid=(n // blk,),
        in_specs=[pl.BlockSpec((blk, d), lambda i: (i, 0)),
                  pl.BlockSpec((2, d), lambda i: (0, 0)),
                  pl.BlockSpec((1, d), lambda i: (0, 0)),
                  pl.BlockSpec((1, d), lambda i: (0, 0))],
        out_specs=pl.BlockSpec((blk, d), lambda i: (i, 0)),
        out_shape=jax.ShapeDtypeStruct((n, d), jnp.float32),
        interpret=_INTERPRET,
    )(x, stats, g.reshape(1, d), b.reshape(1, d))


def _fold_body(a_ref, b_ref, ba_ref, bb_ref, w_ref, bo_ref):
    w_ref[...] = _dot(a_ref[...], b_ref[...])
    bo_ref[...] = _dot(ba_ref[...], b_ref[...]) + bb_ref[...]


def _fold(a, b, ba, bb):
    full = lambda s: pl.BlockSpec(s.shape, lambda: tuple(0 for _ in s.shape))
    return pl.pallas_call(
        _fold_body,
        in_specs=[full(a), full(b), full(ba), full(bb)],
        out_specs=[pl.BlockSpec((a.shape[0], b.shape[1]), lambda: (0, 0)),
                   pl.BlockSpec((1, b.shape[1]), lambda: (0, 0))],
        out_shape=[jax.ShapeDtypeStruct((a.shape[0], b.shape[1]), jnp.float32),
                   jax.ShapeDtypeStruct((1, b.shape[1]), jnp.float32)],
        interpret=_INTERPRET,
    )(a, b, ba, bb)


def _p2_body(hp_ref, g1_ref, g2_ref, wrn_ref, wc_ref, wef_ref, bef_ref,
             weh_ref, beh_ref, h1_ref, n1_ref):
    t1 = _dot(g1_ref[0, :, :], wrn_ref[...])
    h1 = _elu(t1[:, :128] + _dot(g2_ref[0, :, :], wc_ref[...])
              + _dot(hp_ref[...], wef_ref[...]) + bef_ref[...])
    h1_ref[...] = h1
    n1 = _elu(t1[:, 128:] + _dot(h1, weh_ref[...]) + beh_ref[...])
    n1_ref[0, :, :] = n1[:, :128]
    n1_ref[1, :, :] = n1[:, 128:]


def _p2(hprev, g, wrn, wc, wef, bef, weh, beh):
    ne = hprev.shape[1]
    full = lambda a: pl.BlockSpec(a.shape, lambda i: tuple(0 for _ in a.shape))
    return pl.pallas_call(
        _p2_body,
        grid=(E_BLOCKS,),
        in_specs=[pl.BlockSpec((NBLK, ne), lambda i: (i, 0)),
                  pl.BlockSpec((1, NBLK, 128), lambda i: (0, i, 0)),
                  pl.BlockSpec((1, NBLK, 128), lambda i: (1, i, 0)),
                  full(wrn), full(wc),
                  full(wef), full(bef), full(weh), full(beh)],
        out_specs=[pl.BlockSpec((NBLK, 128), lambda i: (i, 0)),
                   pl.BlockSpec((2, NBLK, 128), lambda i: (0, i, 0))],
        out_shape=[jax.ShapeDtypeStruct((E_PAD, 128), jnp.float32),
                   jax.ShapeDtypeStruct((2, E_PAD, 128), jnp.float32)],
        interpret=_INTERPRET,
    )(hprev, g, g, wrn, wc, wef, bef, weh, beh)


def _p3_body(has_u, no, go, *refs):
    if has_u:
        (xn_ref, s0_ref, s1_ref, cnt_ref, b_ref, u_ref,
         wx2_ref, wh2a_ref, wh2b_ref, bv_ref, b1_ref, w2_ref, b2_ref,
         wg1u_ref, wg1g_ref, bg1_ref, wg2_ref, bg2_ref,
         xo_ref, uo_ref, gm_ref, gc_ref) = refs
    else:
        (xn_ref, s0_ref, s1_ref, cnt_ref, b_ref,
         wx2_ref, wh2a_ref, wh2b_ref, bv_ref, b1_ref, w2_ref, b2_ref,
         wg1g_ref, bg1_ref, wg2_ref, bg2_ref,
         xo_ref, uo_ref, gm_ref, gc_ref) = refs
    i = pl.program_id(0)
    cnt_raw = cnt_ref[...]
    cnt = jnp.maximum(cnt_raw, 1.0)
    he = (cnt_raw > 0).astype(jnp.float32)
    hm0 = s0_ref[0, :, :] / cnt
    hm1 = s1_ref[0, :, :] / cnt
    t = _elu(_dot(xn_ref[...], wx2_ref[...]) + _dot(hm0, wh2a_ref[...])
             + _dot(hm1, wh2b_ref[...]) + he * bv_ref[...] + b1_ref[...])
    xnew = _dot(t, w2_ref[...]) + b2_ref[...]
    xo_ref[...] = xnew

    gids = lax.broadcasted_iota(jnp.int32, (1, NGRAPH), 1)
    oh = (b_ref[...] == gids).astype(jnp.float32)
    part = lax.dot_general(oh, xnew, (((0,), (0,)), ((), ())),
                           preferred_element_type=jnp.float32)
    ones = jnp.ones((NBLK, 1), jnp.float32)
    pcnt = lax.dot_general(oh, ones, (((0,), (0,)), ((), ())),
                           preferred_element_type=jnp.float32)

    @pl.when(i == 0)
    def _():
        gm_ref[...] = part
        gc_ref[...] = pcnt

    @pl.when(i > 0)
    def _():
        gm_ref[...] += part
        gc_ref[...] += pcnt

    @pl.when(i == pl.num_programs(0) - 1)
    def _():
        gmean = gm_ref[...] / jnp.maximum(gc_ref[...], 1.0)
        pre = _dot(gmean, wg1g_ref[...]) + bg1_ref[...]
        if has_u:
            pre = pre + _dot(u_ref[...], wg1u_ref[...])
        ug1 = _elu(pre)
        uo_ref[...] = _dot(ug1, wg2_ref[...]) + bg2_ref[...]


def _p3(xn, S, cnt2d, batch2d, u, wx2, wh2a, wh2b, bv, b1, w2, b2,
        wg1u, wg1g, bg1, wg2, bg2):
    has_u = u is not None
    nf = xn.shape[1]
    no = w2.shape[1]
    go = wg2.shape[1]
    full = lambda a: pl.BlockSpec(a.shape, lambda i: tuple(0 for _ in a.shape))
    in_specs = [pl.BlockSpec((NBLK, nf), lambda i: (i, 0)),
                pl.BlockSpec((1, NBLK, 128), lambda i: (0, i, 0)),
                pl.BlockSpec((1, NBLK, 128), lambda i: (1, i, 0)),
                pl.BlockSpec((NBLK, 1), lambda i: (i, 0)),
                pl.BlockSpec((NBLK, 1), lambda i: (i, 0))]
    args = [xn, S, S, cnt2d, batch2d]
    if has_u:
        in_specs.append(full(u))
        args.append(u)
    wargs = [wx2, wh2a, wh2b, bv, b1, w2, b2]
    if has_u:
        wargs.append(wg1u)
    wargs += [wg1g, bg1, wg2, bg2]
    in_specs += [full(a) for a in wargs]
    args += wargs
    body = functools.partial(_p3_body, has_u, no, go)
    return pl.pallas_call(
        body,
        grid=(N_BLOCKS,),
        in_specs=in_specs,
        out_specs=[pl.BlockSpec((NBLK, no), lambda i: (i, 0)),
                   pl.BlockSpec((NGRAPH, go), lambda i: (0, 0))],
        out_shape=[jax.ShapeDtypeStruct((N_PAD, no), jnp.float32),
                   jax.ShapeDtypeStruct((NGRAPH, go), jnp.float32)],
        scratch_shapes=[pltpu.VMEM((NGRAPH, no), jnp.float32),
                        pltpu.VMEM((NGRAPH, 1), jnp.float32)],
        interpret=_INTERPRET,
    )(*args)


def _p5_body(u_ref, w1_ref, b1_ref, w2_ref, b2_ref, o_ref):
    y1 = _elu(_dot(u_ref[...], w1_ref[...]) + b1_ref[...])
    o_ref[...] = _dot(y1, w2_ref[...]) + b2_ref[...]


def _p5(u, w1, b1, w2p, b2p, cblk):
    cpad = w2p.shape[1]
    full = lambda a: pl.BlockSpec(a.shape, lambda j: tuple(0 for _ in a.shape))
    return pl.pallas_call(
        _p5_body,
        grid=(cpad // cblk,),
        in_specs=[full(u), full(w1), full(b1),
                  pl.BlockSpec((w2p.shape[0], cblk), lambda j: (0, j)),
                  pl.BlockSpec((1, cblk), lambda j: (0, j))],
        out_specs=pl.BlockSpec((NGRAPH, cblk), lambda j: (0, j)),
        out_shape=jax.ShapeDtypeStruct((NGRAPH, cpad), jnp.float32),
        interpret=_INTERPRET,
    )(u, w1, b1, w2p, b2p)


def _sc_gather_body(xtab, idx2, g, idx_v, tab_sh, b_v, s0, s1):
    c = lax.axis_index("c")
    s = lax.axis_index("s")
    rb = s * NRT
    pltpu.sync_copy(xtab.at[pl.ds(rb, NRT)], tab_sh.at[pl.ds(rb, NRT)])
    pltpu.sync_copy(idx2.at[c, s], idx_v)
    plsc.subcore_barrier()
    sems = (s0, s1)

    def fire(j, p):
        pltpu.async_copy(tab_sh.at[idx_v.at[j]], b_v.at[p], sems[p])

    def consume(j, p):
        pltpu.make_async_copy(xtab.at[pl.ds(0, KCH)], b_v.at[p],
                              sems[p]).wait()
        pltpu.sync_copy(b_v.at[p], g.at[c, pl.ds(s * ET + j * KCH, KCH)])

    fire(0, 0)
    fire(1, 1)

    @pl.loop(0, SCH - 2, step=2)
    def _(base):
        for b in range(2):
            consume(base + b, b)
            fire(base + b + 2, b)

    for b in range(2):
        consume(SCH - 2 + b, b)


def _sc_scatter_body(with_cnt, *refs):
    if with_cnt:
        (h_st, cidx, zer2, zer1, s_out, cnt_out,
         idx_v, h_v, ones_v, acc, cnt_sh, sh0, sh1) = refs
    else:
        (h_st, cidx, zer2, s_out, idx_v, h_v, acc, sh0, sh1) = refs
    c = lax.axis_index("c")
    s = lax.axis_index("s")
    rb = s * NRT
    shs = (sh0, sh1)
    pltpu.sync_copy(zer2.at[pl.ds(rb, NRT)], acc.at[pl.ds(rb, NRT)])
    if with_cnt:
        @pl.when(c == 0)
        def _():
            pltpu.sync_copy(zer1.at[pl.ds(rb, NRT)], cnt_sh.at[pl.ds(rb, NRT)])
        for l in range(KCH // 16):
            ones_v[pl.ds(l * 16, 16)] = jnp.ones((16,), jnp.float32)
    plsc.subcore_barrier()

    pltpu.sync_copy(cidx.at[s], idx_v)

    def fire(j, p):
        eb = s * ET + j * KCH
        pltpu.async_copy(h_st.at[c, pl.ds(eb, KCH)], h_v.at[p], shs[p])

    def consume(j, p):
        pltpu.make_async_copy(h_st.at[c, pl.ds(0, KCH)], h_v.at[p],
                              shs[p]).wait()
        pltpu.sync_copy(h_v.at[p], acc.at[idx_v.at[j]], add=True)
        if with_cnt:
            @pl.when(c == 0)
            def _():
                pltpu.sync_copy(ones_v, cnt_sh.at[idx_v.at[j]], add=True)

    fire(0, 0)
    fire(1, 1)

    @pl.loop(0, SCH - 2, step=2)
    def _(base):
        for b in range(2):
            consume(base + b, b)
            fire(base + b + 2, b)

    for b in range(2):
        consume(SCH - 2 + b, b)
    plsc.subcore_barrier()

    pltpu.sync_copy(acc.at[pl.ds(rb, NRT)], s_out.at[c, pl.ds(rb, NRT)])
    if with_cnt:
        @pl.when(c == 0)
        def _():
            pltpu.sync_copy(cnt_sh.at[pl.ds(rb, NRT)], cnt_out.at[pl.ds(rb, NRT)])


@functools.cache
def _sc_mesh():
    return plsc.VectorSubcoreMesh(core_axis_name="c", subcore_axis_name="s")


@functools.cache
def _sc_gather_kernel():
    return pl.kernel(
        _sc_gather_body,
        out_type=jax.ShapeDtypeStruct((2, E_PAD, 128), jnp.float32),
        mesh=_sc_mesh(),
        scratch_types=[pltpu.VMEM((SCH, KCH), jnp.int32),
                       pltpu.VMEM_SHARED((N_PAD, 128), jnp.float32),
                       pltpu.VMEM((2, KCH, 128), jnp.float32),
                       pltpu.SemaphoreType.DMA,
                       pltpu.SemaphoreType.DMA],
    )


@functools.cache
def _sc_scatter_kernel(with_cnt):
    if with_cnt:
        return pl.kernel(
            functools.partial(_sc_scatter_body, True),
            out_type=(jax.ShapeDtypeStruct((NC, N_PAD, 128), jnp.float32),
                      jax.ShapeDtypeStruct((N_PAD,), jnp.float32)),
            mesh=_sc_mesh(),
            scratch_types=[pltpu.VMEM((SCH, KCH), jnp.int32),
                           pltpu.VMEM((2, KCH, 128), jnp.float32),
                           pltpu.VMEM((KCH,), jnp.float32),
                           pltpu.VMEM_SHARED((N_PAD, 128), jnp.float32),
                           pltpu.VMEM_SHARED((N_PAD,), jnp.float32),
                           pltpu.SemaphoreType.DMA,
                           pltpu.SemaphoreType.DMA],
        )
    return pl.kernel(
        functools.partial(_sc_scatter_body, False),
        out_type=jax.ShapeDtypeStruct((NC, N_PAD, 128), jnp.float32),
        mesh=_sc_mesh(),
        scratch_types=[pltpu.VMEM((SCH, KCH), jnp.int32),
                       pltpu.VMEM((2, KCH, 128), jnp.float32),
                       pltpu.VMEM_SHARED((N_PAD, 128), jnp.float32),
                       pltpu.SemaphoreType.DMA,
                       pltpu.SemaphoreType.DMA],
    )


def _gather_pallas(xtab, idx2):
    return _sc_gather_kernel()(xtab, idx2)


def _scatter_pallas(h_st, cidx2d, zer2, zer1, with_cnt):
    if with_cnt:
        return _sc_scatter_kernel(True)(h_st, cidx2d, zer2, zer1)
    return _sc_scatter_kernel(False)(h_st, cidx2d, zer2), None


def kernel(x, edge_attr, params, edge_index, batch):
    f32 = jnp.float32
    row = edge_index[0].astype(jnp.int32)
    col = edge_index[1].astype(jnp.int32)
    padi = jnp.full((E_PAD - E_EDGES,), DUMMY, jnp.int32)
    ridx_s = jnp.concatenate([row, padi]).reshape(NS, SCH, KCH)
    cidx_s = jnp.concatenate([col, padi]).reshape(NS, SCH, KCH)
    idx2 = jnp.stack([ridx_s, cidx_s])

    xp = jnp.zeros((N_PAD, x.shape[1]), f32).at[:N_NODES].set(x)
    ep = jnp.zeros((E_PAD, edge_attr.shape[1]), f32).at[:E_EDGES].set(edge_attr)
    bpad = jnp.concatenate(
        [batch.astype(jnp.int32), jnp.full((N_PAD - N_NODES,), NGRAPH, jnp.int32)]
    ).reshape(N_PAD, 1)
    zer2 = jnp.zeros((N_PAD, 128), f32)
    zer1 = jnp.zeros((N_PAD,), f32)

    xs = _stats_x(x)
    es = _stats_e(edge_attr)
    xcur = _bn_apply(xp, xs, params["bn_node"]["g"], params["bn_node"]["b"], NBLK)
    e = _bn_apply(ep, es, params["bn_edge"]["g"], params["bn_edge"]["b"], NBLK)

    u = None
    cnt2d = None
    hprev = e
    w2e_prev = b2e_prev = None
    for li, mp in enumerate(params["metas"]):
        nf = xcur.shape[1]
        w1e, b1e = mp["edge"][0]["w"], mp["edge"][0]["b"]
        w2e, b2e = mp["edge"][1]["w"], mp["edge"][1]["b"]
        w1n, b1n = mp["node1"][0]["w"], mp["node1"][0]["b"]
        w2n1, b2n1 = mp["node1"][1]["w"], mp["node1"][1]["b"]
        wxr, wxc, wee = w1e[:nf], w1e[nf:2 * nf], w1e[2 * nf:]
        wnx, whh = w1n[:nf], w1n[nf:]
        wrn = jnp.concatenate([wxr, wnx], axis=1)
        if nf % 128:
            xg = jnp.zeros((N_PAD, 128), f32).at[:, :nf].set(xcur)
            wrn = jnp.zeros((128, wrn.shape[1]), f32).at[:nf].set(wrn)
            wxc = jnp.zeros((128, wxc.shape[1]), f32).at[:nf].set(wxc)
        else:
            xg = xcur

        g = _gather_pallas(xg, idx2)

        if li == 0:
            wef, bef = wee, b1e.reshape(1, -1)
        else:
            wef, bef = _fold(w2e_prev, wee, b2e_prev.reshape(1, -1),
                             b1e.reshape(1, -1))
        weh, beh = _fold(w2e, whh, b2e.reshape(1, -1), b1n.reshape(1, -1))
        h1, n1_st = _p2(hprev, g, wrn, wxc, wef, bef, weh, beh)
        S, cnt = _scatter_pallas(n1_st, cidx_s, zer2, zer1, li == 0)
        if li == 0:
            cnt2d = cnt.reshape(N_PAD, 1)

        w1n2, b1n2 = mp["node2"][0]["w"], mp["node2"][0]["b"]
        w2n2, b2n2 = mp["node2"][1]["w"], mp["node2"][1]["b"]
        wx2, wh2 = w1n2[:nf], w1n2[nf:]
        wf, bv = _fold(w2n1, wh2, b2n1.reshape(1, -1),
                       jnp.zeros((1, wh2.shape[1]), f32))
        wh2a, wh2b = wf[:128], wf[128:]
        w1g, b1g = mp["glob"][0]["w"], mp["glob"][0]["b"]
        w2g, b2g = mp["glob"][1]["w"], mp["glob"][1]["b"]
        ng = 0 if u is None else u.shape[1]
        wg1u = w1g[:ng] if ng else None
        wg1g = w1g[ng:]
        xcur, u = _p3(xcur, S, cnt2d, bpad, u, wx2, wh2a, wh2b, bv,
                      b1n2.reshape(1, -1), w2n2, b2n2.reshape(1, -1),
                      wg1u, wg1g, b1g.reshape(1, -1), w2g, b2g.reshape(1, -1))
        hprev = h1
        w2e_prev, b2e_prev = w2e, b2e

    w1, b1 = params["lin1"]["w"], params["lin1"]["b"]
    w2, b2 = params["lin2"]["w"], params["lin2"]["b"]
    cout = w2.shape[1]
    cpad = 14592
    w2p = jnp.zeros((w2.shape[0], cpad), f32).at[:, :cout].set(w2)
    b2p = jnp.zeros((1, cpad), f32).at[0, :cout].set(b2)
    y = _p5(u, w1, b1.reshape(1, -1), w2p, b2p, 2432)
    return (u, y[:, :cout])
